# Initial kernel scaffold; baseline (speedup 1.0000x reference)
import jax
import jax.numpy as jnp
from jax.experimental import pallas as pl


def kernel(x, emb, W1, b1, W2, b2):
    raise NotImplementedError("write your pallas kernel here")


# speedup vs baseline: 12.1607x; 12.1607x over previous
import functools

import jax
import jax.numpy as jnp
from jax import lax
from jax.experimental import pallas as pl
from jax.experimental.pallas import tpu as pltpu
from jax.experimental.pallas import tpu_sc as plsc

B = 4096
L = 200
V = 100000
D = 304
LP = 208
CH = LP // 2
NW = 32
SPW = B // NW
DJ = D // 16


def _transform_body(e_ref, w_ref, b_ref, t_ref):
  t_ref[...] = (
      jnp.dot(e_ref[...], w_ref[...], preferred_element_type=jnp.float32)
      + b_ref[...]
  )


def _transform_table(emb_p, w1t_p, b1_p):
  mblk = 2000
  return pl.pallas_call(
      _transform_body,
      grid=(V // mblk,),
      in_specs=[
          pl.BlockSpec((mblk, D), lambda i: (i, 0)),
          pl.BlockSpec((D, D), lambda i: (0, 0)),
          pl.BlockSpec((1, D), lambda i: (0, 0)),
      ],
      out_specs=pl.BlockSpec((mblk, D), lambda i: (i, 0)),
      out_shape=jax.ShapeDtypeStruct((V, D), jnp.float32),
  )(emb_p, w1t_p, b1_p)


def _pool_body(t_hbm, x_hbm, out_hbm, idx_v, rows_v, out_v, sem0, sem1):
  wid = lax.axis_index("s") * 2 + lax.axis_index("c")
  ibase = pl.multiple_of(wid * (SPW * LP), 8)
  obase = pl.multiple_of(wid * (SPW * D), 8)

  pltpu.sync_copy(x_hbm.at[pl.ds(ibase, SPW * LP)], idx_v)

  sems = (sem0, sem1)

  def issue(s, c):
    off = pl.multiple_of(s * LP + c * CH, 8)
    pltpu.async_copy(
        t_hbm.at[idx_v.at[pl.ds(off, CH)]], rows_v.at[c], sems[c]
    )

  def wait(c):
    pltpu.make_async_copy(
        t_hbm.at[idx_v.at[pl.ds(0, CH)]], rows_v.at[c], sems[c]
    ).wait()

  issue(0, 0)
  issue(0, 1)

  RUN = 4

  def body(s, carry):
    accs = tuple(jnp.zeros((16,), jnp.float32) for _ in range(DJ))
    for c in range(2):
      wait(c)

      def rbody(i, a, c=c):
        a = list(a)
        for dr in range(RUN):
          r = i * RUN + dr
          for j in range(DJ):
            a[j] = jnp.maximum(a[j], rows_v[c, r, pl.ds(16 * j, 16)])
        return tuple(a)

      accs = lax.fori_loop(0, CH // RUN, rbody, accs)

      @pl.when(s + 1 < SPW)
      def _():
        issue(s + 1, c)

    off = s * D
    for j in range(DJ):
      out_v[pl.ds(off + 16 * j, 16)] = accs[j]
    return carry

  lax.fori_loop(0, SPW, body, 0)
  pltpu.sync_copy(out_v, out_hbm.at[pl.ds(obase, SPW * D)])


def _pool(table, x_flat):
  mesh = plsc.VectorSubcoreMesh(
      core_axis_name="c", subcore_axis_name="s", num_cores=2, num_subcores=16
  )
  k = pl.kernel(
      _pool_body,
      out_type=jax.ShapeDtypeStruct((B * D,), jnp.float32),
      mesh=mesh,
      scratch_types=[
          pltpu.VMEM((SPW * LP,), jnp.int32),
          pltpu.VMEM((2, CH, D), jnp.float32),
          pltpu.VMEM((SPW * D,), jnp.float32),
          pltpu.SemaphoreType.DMA,
          pltpu.SemaphoreType.DMA,
      ],
      compiler_params=pltpu.CompilerParams(use_tc_tiling_on_sc=False),
  )
  return k(table, x_flat)


def _fc2_body(h_ref, w_ref, b_ref, o_ref):
  o_ref[...] = (
      jnp.dot(h_ref[...], w_ref[...], preferred_element_type=jnp.float32)
      + b_ref[...]
  )


def _fc2(pool, w2t_p, b2_p):
  mblk = 1024
  return pl.pallas_call(
      _fc2_body,
      grid=(B // mblk,),
      in_specs=[
          pl.BlockSpec((mblk, D), lambda i: (i, 0)),
          pl.BlockSpec((D, 128), lambda i: (0, 0)),
          pl.BlockSpec((1, 128), lambda i: (0, 0)),
      ],
      out_specs=pl.BlockSpec((mblk, 128), lambda i: (i, 0)),
      out_shape=jax.ShapeDtypeStruct((B, 128), jnp.float32),
  )(pool, w2t_p, b2_p)


@jax.jit
def kernel(x, emb, W1, b1, W2, b2):
  emb_p = jnp.pad(emb, ((0, 0), (0, D - 300)))
  w1t_p = jnp.pad(W1.T, ((0, D - 300), (0, D - 300)))
  b1_p = jnp.pad(b1, (0, D - 300)).reshape(1, D)

  table = _transform_table(emb_p, w1t_p, b1_p)

  x_i = x.astype(jnp.int32)
  x_pad = jnp.concatenate([x_i, x_i[:, :8]], axis=1).reshape(-1)

  pool = _pool(table, x_pad).reshape(B, D)

  w2t_p = jnp.pad(W2.T, ((0, D - 300), (0, 126)))
  b2_p = jnp.pad(b2, (0, 126)).reshape(1, 128)
  out = _fc2(pool, w2t_p, b2_p)
  return out[:, :2]
